# baseline (speedup 1.0000x reference)
import functools

import jax
import jax.numpy as jnp
from jax import lax
from jax.experimental import pallas as pl
from jax.experimental.pallas import tpu as pltpu
from jax.experimental.pallas import tpu_sc as plsc

N = 100000
T = 28
D = 64

RB = 512
NB = (N + RB - 1) // RB

C = 384
NCHUNK = (N + C - 1) // C
LAST_START = N - C
IDX_ROWS = C // 128


def _argmax_rows(x_ref, idx_ref):
    xb = x_ref[...]
    m = jnp.max(xb, axis=1, keepdims=True)
    cols = lax.broadcasted_iota(jnp.int32, xb.shape, 1)
    cand = jnp.where(xb == m, cols, T)
    idx_ref[...] = jnp.min(cand, axis=1)


def _tc_argmax(x):
    return pl.pallas_call(
        _argmax_rows,
        grid=(NB,),
        in_specs=[pl.BlockSpec((RB, T), lambda i: (i, 0))],
        out_specs=pl.BlockSpec((RB,), lambda i: (i,)),
        out_shape=jax.ShapeDtypeStruct((N,), jnp.int32),
    )(x)


def kernel(x, W):
    info = plsc.get_sparse_core_info()
    nc, ns = info.num_cores, info.num_subcores
    nw = nc * ns
    cpw = (NCHUNK + nw - 1) // nw

    mesh = plsc.VectorSubcoreMesh(core_axis_name="c", subcore_axis_name="s")

    @functools.partial(
        pl.kernel,
        mesh=mesh,
        out_type=jax.ShapeDtypeStruct((N, 128), jnp.float32),
        scratch_types=[
            pltpu.VMEM((2 * C,), jnp.int32),
            pltpu.VMEM((2 * C, 128), jnp.float32),
            pltpu.VMEM_SHARED((T, 128), jnp.float32),
            pltpu.SemaphoreType.DMA,
            pltpu.SemaphoreType.DMA,
            pltpu.SemaphoreType.DMA,
        ],
        compiler_params=pltpu.CompilerParams(
            needs_layout_passes=False, use_tc_tiling_on_sc=False),
    )
    def run(idx_hbm, w_hbm, out_hbm, idx_v, rows_v, w_v,
            sem_x, sem_g, sem_w):
        wid = lax.axis_index("s") * nc + lax.axis_index("c")

        def chunk_start(k):
            return jnp.minimum(k * C, LAST_START)

        def start_idx_load(t):
            k = wid + t * nw
            b = t % 2
            pltpu.async_copy(
                idx_hbm.at[pl.ds(chunk_start(k), C)],
                idx_v.at[pl.ds(b * C, C)],
                sem_x)

        def wait_idx_load(t):
            b = t % 2
            pltpu.make_async_copy(
                idx_hbm.at[pl.ds(0, C)],
                idx_v.at[pl.ds(b * C, C)],
                sem_x).wait()

        def wait_out_write():
            pltpu.make_async_copy(
                rows_v.at[pl.ds(0, C)],
                out_hbm.at[pl.ds(0, C)],
                sem_w).wait()

        def chunk_body(t, carry):
            k = wid + t * nw
            b = t % 2

            @pl.when(k < NCHUNK)
            def _do():
                wait_idx_load(t)

                @pl.when(k + nw < NCHUNK)
                def _prefetch():
                    start_idx_load(t + 1)

                copies = [
                    pltpu.async_copy(
                        w_v.at[idx_v.at[pl.ds(b * C + j * 128, 128)]],
                        rows_v.at[pl.ds(b * C + j * 128, 128)],
                        sem_g)
                    for j in range(IDX_ROWS)
                ]
                for cp in copies:
                    cp.wait()

                @pl.when(t >= 1)
                def _drain_prev_write():
                    wait_out_write()

                pltpu.async_copy(
                    rows_v.at[pl.ds(b * C, C)],
                    out_hbm.at[pl.ds(chunk_start(k), C)],
                    sem_w)

            return carry

        @pl.when(lax.axis_index("s") == 0)
        def _stage_table():
            pltpu.sync_copy(w_hbm, w_v)

        plsc.subcore_barrier()
        start_idx_load(0)
        lax.fori_loop(0, cpw, chunk_body, 0)
        wait_out_write()

    idx = _tc_argmax(x)
    out128 = run(idx, jnp.pad(W, ((0, 0), (0, 128 - D))))
    return out128[:, :D]


# speedup vs baseline: 1.6559x; 1.6559x over previous
import functools

import jax
import jax.numpy as jnp
from jax import lax
from jax.experimental import pallas as pl
from jax.experimental.pallas import tpu as pltpu
from jax.experimental.pallas import tpu_sc as plsc

N = 100000
T = 28
D = 64
C = 384
NCHUNK = (N + C - 1) // C
LAST_START = N - C
GROUPS = C // 16
IDX_ROWS = C // 128


def _argmax16(vals):
    pairs = list(vals)
    while len(pairs) > 1:
        nxt = []
        for i in range(0, len(pairs) - 1, 2):
            (va, ia), (vb, ib) = pairs[i], pairs[i + 1]
            m = vb > va
            nxt.append((jnp.where(m, vb, va), jnp.where(m, ib, ia)))
        if len(pairs) % 2:
            nxt.append(pairs[-1])
        pairs = nxt
    return pairs[0][1]


def kernel(x, W):
    info = plsc.get_sparse_core_info()
    nc, ns = info.num_cores, info.num_subcores
    nw = nc * ns
    cpw = (NCHUNK + nw - 1) // nw

    mesh = plsc.VectorSubcoreMesh(core_axis_name="c", subcore_axis_name="s")

    @functools.partial(
        pl.kernel,
        mesh=mesh,
        out_type=jax.ShapeDtypeStruct((N, 128), jnp.float32),
        scratch_types=[
            pltpu.VMEM((2 * C * T // 128, 128), jnp.float32),
            pltpu.VMEM((2 * IDX_ROWS * 128,), jnp.int32),
            pltpu.VMEM((2 * C, 128), jnp.float32),
            pltpu.VMEM_SHARED((T, 128), jnp.float32),
            pltpu.SemaphoreType.DMA,
            pltpu.SemaphoreType.DMA,
            pltpu.SemaphoreType.DMA,
        ],
        compiler_params=pltpu.CompilerParams(
            needs_layout_passes=False, use_tc_tiling_on_sc=False),
    )
    def run(x_hbm, w_hbm, out_hbm, x_v, idx_v, rows_v, w_v,
            sem_x, sem_g, sem_w):
        wid = lax.axis_index("s") * nc + lax.axis_index("c")
        lane = lax.broadcasted_iota(jnp.int32, (16,), 0)

        def chunk_start(k):
            return jnp.minimum(k * C, LAST_START)

        XROWS = C * T // 128

        def start_x_load(t):
            k = wid + t * nw
            b = t % 2
            pltpu.async_copy(
                x_hbm.at[pl.ds(chunk_start(k) * T // 128, XROWS)],
                x_v.at[pl.ds(b * XROWS, XROWS)],
                sem_x)

        def wait_x_load(t):
            b = t % 2
            pltpu.make_async_copy(
                x_hbm.at[pl.ds(0, XROWS)],
                x_v.at[pl.ds(b * XROWS, XROWS)],
                sem_x).wait()

        def wait_out_write():
            pltpu.make_async_copy(
                rows_v.at[pl.ds(0, C)],
                out_hbm.at[pl.ds(0, C)],
                sem_w).wait()

        def chunk_body(t, carry):
            k = wid + t * nw
            b = t % 2

            @pl.when(k < NCHUNK)
            def _do():
                wait_x_load(t)

                @pl.when(k + nw < NCHUNK)
                def _prefetch():
                    start_x_load(t + 1)

                xbase = b * C * T
                for g in range(GROUPS):
                    base = xbase + (g * 16 + lane) * T
                    vals = []
                    for c in range(T):
                        f = base + jnp.full((16,), c, jnp.int32)
                        vals.append((plsc.load_gather(
                            x_v, [f >> 7, f & 127]),
                            jnp.full((16,), c, jnp.int32)))
                    bidx = _argmax16(vals)
                    idx_v[pl.ds(b * IDX_ROWS * 128 + g * 16, 16)] = bidx
                copies = [
                    pltpu.async_copy(
                        w_v.at[idx_v.at[pl.ds(
                            b * IDX_ROWS * 128 + j * 128, 128)]],
                        rows_v.at[pl.ds(b * C + j * 128, 128)],
                        sem_g)
                    for j in range(IDX_ROWS)
                ]
                for cp in copies:
                    cp.wait()

                @pl.when(t >= 1)
                def _drain_prev_write():
                    wait_out_write()

                pltpu.async_copy(
                    rows_v.at[pl.ds(b * C, C)],
                    out_hbm.at[pl.ds(chunk_start(k), C)],
                    sem_w)

            return carry

        @pl.when(lax.axis_index("s") == 0)
        def _stage_table():
            pltpu.sync_copy(w_hbm, w_v)

        plsc.subcore_barrier()
        start_x_load(0)
        lax.fori_loop(0, cpw, chunk_body, 0)
        wait_out_write()

    out128 = run(x.reshape(N * T // 128, 128),
                 jnp.pad(W, ((0, 0), (0, 128 - D))))
    return out128[:, :D]


# speedup vs baseline: 1.7271x; 1.0430x over previous
import functools

import jax
import jax.numpy as jnp
from jax import lax
from jax.experimental import pallas as pl
from jax.experimental.pallas import tpu as pltpu
from jax.experimental.pallas import tpu_sc as plsc

N = 100000
T = 28
D = 64
C = 256
NCHUNK = (N + C - 1) // C
LAST_START = N - C
GROUPS = C // 16
IDX_ROWS = C // 128


def _argmax16(vals):
    pairs = list(vals)
    while len(pairs) > 1:
        nxt = []
        for i in range(0, len(pairs) - 1, 2):
            (va, ia), (vb, ib) = pairs[i], pairs[i + 1]
            m = vb > va
            nxt.append((jnp.where(m, vb, va), jnp.where(m, ib, ia)))
        if len(pairs) % 2:
            nxt.append(pairs[-1])
        pairs = nxt
    return pairs[0][1]


def kernel(x, W):
    info = plsc.get_sparse_core_info()
    nc, ns = info.num_cores, info.num_subcores
    nw = nc * ns
    cpw = (NCHUNK + nw - 1) // nw

    mesh = plsc.VectorSubcoreMesh(core_axis_name="c", subcore_axis_name="s")

    @functools.partial(
        pl.kernel,
        mesh=mesh,
        out_type=jax.ShapeDtypeStruct((N, 128), jnp.float32),
        scratch_types=[
            pltpu.VMEM((2 * C * T // 128, 128), jnp.float32),
            pltpu.VMEM((2 * IDX_ROWS * 128,), jnp.int32),
            pltpu.VMEM((2 * C, 128), jnp.float32),
            pltpu.VMEM_SHARED((T, 128), jnp.float32),
            pltpu.SemaphoreType.DMA,
            pltpu.SemaphoreType.DMA,
            pltpu.SemaphoreType.DMA,
        ],
        compiler_params=pltpu.CompilerParams(
            needs_layout_passes=False, use_tc_tiling_on_sc=False),
    )
    def run(x_hbm, w_hbm, out_hbm, x_v, idx_v, rows_v, w_v,
            sem_x, sem_g, sem_w):
        wid = lax.axis_index("s") * nc + lax.axis_index("c")
        lane = lax.broadcasted_iota(jnp.int32, (16,), 0)

        def chunk_start(k):
            return jnp.minimum(k * C, LAST_START)

        XROWS = C * T // 128

        def start_x_load(t):
            k = wid + t * nw
            b = t % 2
            pltpu.async_copy(
                x_hbm.at[pl.ds(chunk_start(k) * T // 128, XROWS)],
                x_v.at[pl.ds(b * XROWS, XROWS)],
                sem_x)

        def wait_x_load(t):
            b = t % 2
            pltpu.make_async_copy(
                x_hbm.at[pl.ds(0, XROWS)],
                x_v.at[pl.ds(b * XROWS, XROWS)],
                sem_x).wait()

        def wait_out_write():
            pltpu.make_async_copy(
                rows_v.at[pl.ds(0, C)],
                out_hbm.at[pl.ds(0, C)],
                sem_w).wait()

        def chunk_body(t, carry):
            k = wid + t * nw
            b = t % 2

            @pl.when(k < NCHUNK)
            def _do():
                wait_x_load(t)

                @pl.when(k + nw < NCHUNK)
                def _prefetch():
                    start_x_load(t + 1)

                xbase = b * C * T
                for g in range(GROUPS):
                    base = xbase + (g * 16 + lane) * T
                    vals = []
                    for c in range(T):
                        f = base + jnp.full((16,), c, jnp.int32)
                        vals.append((plsc.load_gather(
                            x_v, [f >> 7, f & 127]),
                            jnp.full((16,), c, jnp.int32)))
                    bidx = _argmax16(vals)
                    idx_v[pl.ds(b * IDX_ROWS * 128 + g * 16, 16)] = bidx
                copies = [
                    pltpu.async_copy(
                        w_v.at[idx_v.at[pl.ds(
                            b * IDX_ROWS * 128 + j * 128, 128)]],
                        rows_v.at[pl.ds(b * C + j * 128, 128)],
                        sem_g)
                    for j in range(IDX_ROWS)
                ]
                for cp in copies:
                    cp.wait()

                @pl.when(t >= 1)
                def _drain_prev_write():
                    wait_out_write()

                pltpu.async_copy(
                    rows_v.at[pl.ds(b * C, C)],
                    out_hbm.at[pl.ds(chunk_start(k), C)],
                    sem_w)

            return carry

        @pl.when(lax.axis_index("s") == 0)
        def _stage_table():
            pltpu.sync_copy(w_hbm, w_v)

        plsc.subcore_barrier()
        start_x_load(0)
        lax.fori_loop(0, cpw, chunk_body, 0)
        wait_out_write()

    out128 = run(x.reshape(N * T // 128, 128),
                 jnp.pad(W, ((0, 0), (0, 128 - D))))
    return out128[:, :D]
